# Initial kernel scaffold; baseline (speedup 1.0000x reference)
import jax
import jax.numpy as jnp
from jax.experimental import pallas as pl


def kernel(x):
    raise NotImplementedError("write your pallas kernel here")


# speedup vs baseline: 20.5264x; 20.5264x over previous
import functools

import jax
import jax.numpy as jnp
from jax import lax
from jax.experimental import pallas as pl
from jax.experimental.pallas import tpu as pltpu
from jax.experimental.pallas import tpu_sc as plsc

_NBINS = 16
_SLOTS = 17
_K = 1.875
_C1 = 8.5
_S = 64
_P = 256
_NW = 32


def _make_sc(B, CB):
    C = CB * _P
    BPW = B // _NW
    NCH = BPW // CB
    NPAIR = NCH // 2
    assert B % _NW == 0 and BPW % CB == 0 and NCH % 2 == 0

    def zero_hist(hist_ref):
        z16 = jnp.zeros((16,), jnp.float32)

        @plsc.parallel_loop(0, C * _SLOTS // 16, unroll=8)
        def _zloop(i):
            hist_ref[pl.ds(i * 16, 16)] = z16

    def bin_chunk(in_ref, hist_ref):
        ones = jnp.ones((16,), jnp.float32)
        iota = lax.iota(jnp.int32, 16)

        def sbody(slab, c):
            bloc = slab // _S
            s = slab % _S
            rows0 = bloc * _P + iota

            @plsc.parallel_loop(0, _P // 16, unroll=8)
            def _ploop(pg):
                v = in_ref[bloc, s, pl.ds(pg * 16, 16)]
                t = v * _K + _C1
                t = jnp.minimum(jnp.maximum(t, 0.0), 16.5)
                b = t.astype(jnp.int32) * C + (rows0 + pg * 16)
                plsc.addupdate_scatter(hist_ref, [b], ones)

            return c

        lax.fori_loop(0, CB * _S, sbody, 0)

    def readout_chunk(hist_ref, out_ref):
        @plsc.parallel_loop(0, C // 16, unroll=1)
        def _gloop(g):
            bloc = g // (_P // 16)
            pp = (g % (_P // 16)) * 16
            c0 = hist_ref[pl.ds(g * 16, 16)]
            rinv = 1.0 / (64.0 - c0)
            for i in range(1, _SLOTS):
                cj = hist_ref[pl.ds(i * C + g * 16, 16)]
                out_ref[bloc, i - 1, pl.ds(pp, 16)] = cj * rinv

    def sc_body(x_hbm, out_hbm, in0, in1, ob0, ob1, hist,
                sem_i0, sem_i1, sem_o0, sem_o1):
        cid = lax.axis_index("c")
        sid = lax.axis_index("s")
        wid = sid * 2 + cid
        wbase = wid * BPW

        ins = (in0, in1)
        sis = (sem_i0, sem_i1)
        obs = (ob0, ob1)
        sos = (sem_o0, sem_o1)

        def in_copy(bbase, buf, sem):
            return pltpu.make_async_copy(
                x_hbm.at[pl.ds(bbase, CB)], buf, sem)

        def out_copy(bbase, buf, sem):
            return pltpu.make_async_copy(
                buf, out_hbm.at[pl.ds(bbase, CB)], sem)

        in_copy(wbase, in0, sem_i0).start()

        def pair_body(p, carry):
            base0 = wbase + 2 * p * CB

            for b in range(2):
                base = base0 + b * CB
                nxt = base + CB

                if b == 0:
                    in_copy(nxt, ins[1], sis[1]).start()
                else:
                    @pl.when(p < NPAIR - 1)
                    def _prefetch():
                        in_copy(nxt, ins[0], sis[0]).start()

                in_copy(base, ins[b], sis[b]).wait()
                zero_hist(hist)
                bin_chunk(ins[b], hist)

                @pl.when(p > 0)
                def _drain_prev_out():
                    out_copy(base, obs[b], sos[b]).wait()

                readout_chunk(hist, obs[b])
                out_copy(base, obs[b], sos[b]).start()
            return carry

        lax.fori_loop(0, NPAIR, pair_body, 0)

        last0 = wbase + (NCH - 2) * CB
        out_copy(last0, ob0, sem_o0).wait()
        out_copy(last0 + CB, ob1, sem_o1).wait()

    mesh = plsc.VectorSubcoreMesh(
        core_axis_name="c", subcore_axis_name="s",
        num_cores=2, num_subcores=16)
    return functools.partial(
        pl.kernel,
        mesh=mesh,
        out_type=jax.ShapeDtypeStruct((B, _NBINS, _P), jnp.float32),
        scratch_types=[
            pltpu.VMEM((CB, _S, _P), jnp.float32),
            pltpu.VMEM((CB, _S, _P), jnp.float32),
            pltpu.VMEM((CB, _NBINS, _P), jnp.float32),
            pltpu.VMEM((CB, _NBINS, _P), jnp.float32),
            pltpu.VMEM((C * _SLOTS,), jnp.float32),
            pltpu.SemaphoreType.DMA,
            pltpu.SemaphoreType.DMA,
            pltpu.SemaphoreType.DMA,
            pltpu.SemaphoreType.DMA,
        ],
        compiler_params=pltpu.CompilerParams(
            needs_layout_passes=False, use_tc_tiling_on_sc=True),
    )(sc_body)


def kernel(x):
    batch, patches, seq = x.shape
    xt = jnp.transpose(x, (0, 2, 1))
    run = _make_sc(batch, 2)
    zt = run(xt)
    return jnp.transpose(zt, (0, 2, 1))


# speedup vs baseline: 26.6853x; 1.3000x over previous
import functools

import jax
import jax.numpy as jnp
from jax import lax
from jax.experimental import pallas as pl
from jax.experimental.pallas import tpu as pltpu
from jax.experimental.pallas import tpu_sc as plsc

_NBINS = 16
_SLOTS = 17
_K = 1.875
_C1 = 8.5
_S = 64
_P = 256
_NW = 32


def _make_sc(B, CB):
    C = CB * _P
    BPW = B // _NW
    NCH = BPW // CB
    NPAIR = NCH // 2
    assert B % _NW == 0 and BPW % CB == 0 and NCH % 2 == 0

    def zero_hist(hist_ref):
        z16 = jnp.zeros((16,), jnp.float32)

        @plsc.parallel_loop(0, C * _SLOTS // 16, unroll=8)
        def _zloop(i):
            hist_ref[pl.ds(i * 16, 16)] = z16

    def bin_chunk(in_ref, hist_ref):
        ones = jnp.ones((16,), jnp.float32)
        iota = lax.iota(jnp.int32, 16)

        @plsc.parallel_loop(0, CB * _S, unroll=2)
        def _sloop(slab):
            bloc = slab // _S
            s = slab % _S
            rows0 = bloc * _P + iota

            for pg in range(_P // 16):
                v = in_ref[bloc, s, pl.ds(pg * 16, 16)]
                t = v * _K + _C1
                t = jnp.minimum(jnp.maximum(t, 0.0), 16.5)
                b = t.astype(jnp.int32) * C + (rows0 + pg * 16)
                plsc.addupdate_scatter(hist_ref, [b], ones)

    def readout_chunk(hist_ref, out_ref):
        @plsc.parallel_loop(0, C // 16, unroll=2)
        def _gloop(g):
            bloc = g // (_P // 16)
            pp = (g % (_P // 16)) * 16
            c0 = hist_ref[pl.ds(g * 16, 16)]
            rinv = 1.0 / (64.0 - c0)
            for i in range(1, _SLOTS):
                cj = hist_ref[pl.ds(i * C + g * 16, 16)]
                out_ref[bloc, i - 1, pl.ds(pp, 16)] = cj * rinv

    def sc_body(x_hbm, out_hbm, in0, in1, ob0, ob1, hist,
                sem_i0, sem_i1, sem_o0, sem_o1):
        cid = lax.axis_index("c")
        sid = lax.axis_index("s")
        wid = sid * 2 + cid
        wbase = wid * BPW

        ins = (in0, in1)
        sis = (sem_i0, sem_i1)
        obs = (ob0, ob1)
        sos = (sem_o0, sem_o1)

        def in_copy(bbase, buf, sem):
            return pltpu.make_async_copy(
                x_hbm.at[pl.ds(bbase, CB)], buf, sem)

        def out_copy(bbase, buf, sem):
            return pltpu.make_async_copy(
                buf, out_hbm.at[pl.ds(bbase, CB)], sem)

        in_copy(wbase, in0, sem_i0).start()

        def pair_body(p, carry):
            base0 = wbase + 2 * p * CB

            for b in range(2):
                base = base0 + b * CB
                nxt = base + CB

                if b == 0:
                    in_copy(nxt, ins[1], sis[1]).start()
                else:
                    @pl.when(p < NPAIR - 1)
                    def _prefetch():
                        in_copy(nxt, ins[0], sis[0]).start()

                in_copy(base, ins[b], sis[b]).wait()
                zero_hist(hist)
                bin_chunk(ins[b], hist)

                @pl.when(p > 0)
                def _drain_prev_out():
                    out_copy(base, obs[b], sos[b]).wait()

                readout_chunk(hist, obs[b])
                out_copy(base, obs[b], sos[b]).start()
            return carry

        lax.fori_loop(0, NPAIR, pair_body, 0)

        last0 = wbase + (NCH - 2) * CB
        out_copy(last0, ob0, sem_o0).wait()
        out_copy(last0 + CB, ob1, sem_o1).wait()

    mesh = plsc.VectorSubcoreMesh(
        core_axis_name="c", subcore_axis_name="s",
        num_cores=2, num_subcores=16)
    return functools.partial(
        pl.kernel,
        mesh=mesh,
        out_type=jax.ShapeDtypeStruct((B, _NBINS, _P), jnp.float32),
        scratch_types=[
            pltpu.VMEM((CB, _S, _P), jnp.float32),
            pltpu.VMEM((CB, _S, _P), jnp.float32),
            pltpu.VMEM((CB, _NBINS, _P), jnp.float32),
            pltpu.VMEM((CB, _NBINS, _P), jnp.float32),
            pltpu.VMEM((C * _SLOTS,), jnp.float32),
            pltpu.SemaphoreType.DMA,
            pltpu.SemaphoreType.DMA,
            pltpu.SemaphoreType.DMA,
            pltpu.SemaphoreType.DMA,
        ],
        compiler_params=pltpu.CompilerParams(
            needs_layout_passes=False, use_tc_tiling_on_sc=True),
    )(sc_body)


def kernel(x):
    batch, patches, seq = x.shape
    xt = jnp.transpose(x, (0, 2, 1))
    run = _make_sc(batch, 2)
    zt = run(xt)
    return jnp.transpose(zt, (0, 2, 1))


# speedup vs baseline: 33.8120x; 1.2671x over previous
import functools

import jax
import jax.numpy as jnp
from jax import lax
from jax.experimental import pallas as pl
from jax.experimental.pallas import tpu as pltpu
from jax.experimental.pallas import tpu_sc as plsc

_NBINS = 16
_SLOTS = 17
_K = 1.875
_S = 64
_P = 256
_NW = 32


def _make_sc(B, CB):
    C = CB * _P
    BPW = B // _NW
    NCH = BPW // CB
    NPAIR = NCH // 2
    assert B % _NW == 0 and BPW % CB == 0 and NCH % 2 == 0

    def zero_hist(hist_ref):
        z16 = jnp.zeros((16,), jnp.float32)

        @plsc.parallel_loop(0, C * _SLOTS // 16, unroll=8)
        def _zloop(i):
            hist_ref[pl.ds(i * 16, 16)] = z16

    def bin_chunk(in_ref, hist_ref):
        ones = jnp.ones((16,), jnp.float32)
        iota = lax.iota(jnp.int32, 16)

        assert C == 512
        @plsc.parallel_loop(0, CB * _S, unroll=4)
        def _sloop(slab):
            bloc = slab // _S
            s = slab % _S
            rows0 = (bloc * _P + iota) ^ (-2147483648)

            for pg in range(_P // 16):
                v = in_ref[bloc, s, pl.ds(pg * 16, 16)]
                t = v * _K + 12582920.0
                t = jnp.minimum(jnp.maximum(t, 12582912.0), 12582928.0)
                ti = jax.lax.bitcast_convert_type(t, jnp.int32)
                b = ti * C + (rows0 + pg * 16)
                plsc.addupdate_scatter(hist_ref, [b], ones)

    def readout_chunk(hist_ref, out_ref):
        z16 = jnp.zeros((16,), jnp.float32)

        @plsc.parallel_loop(0, C // 16, unroll=2)
        def _gloop(g):
            bloc = g // (_P // 16)
            pp = (g % (_P // 16)) * 16
            c0 = hist_ref[pl.ds(g * 16, 16)]
            hist_ref[pl.ds(g * 16, 16)] = z16
            rinv = 1.0 / (64.0 - c0)
            for i in range(1, _SLOTS):
                cj = hist_ref[pl.ds(i * C + g * 16, 16)]
                hist_ref[pl.ds(i * C + g * 16, 16)] = z16
                out_ref[bloc, i - 1, pl.ds(pp, 16)] = cj * rinv

    def sc_body(x_hbm, out_hbm, in0, in1, ob0, ob1, hist,
                sem_i0, sem_i1, sem_o0, sem_o1):
        cid = lax.axis_index("c")
        sid = lax.axis_index("s")
        wid = sid * 2 + cid
        wbase = wid * BPW

        ins = (in0, in1)
        sis = (sem_i0, sem_i1)
        obs = (ob0, ob1)
        sos = (sem_o0, sem_o1)

        def in_copy(bbase, buf, sem):
            return pltpu.make_async_copy(
                x_hbm.at[pl.ds(bbase, CB)], buf, sem)

        def out_copy(bbase, buf, sem):
            return pltpu.make_async_copy(
                buf, out_hbm.at[pl.ds(bbase, CB)], sem)

        in_copy(wbase, in0, sem_i0).start()
        zero_hist(hist)

        def pair_body(p, carry):
            base0 = wbase + 2 * p * CB

            for b in range(2):
                base = base0 + b * CB
                nxt = base + CB

                if b == 0:
                    in_copy(nxt, ins[1], sis[1]).start()
                else:
                    @pl.when(p < NPAIR - 1)
                    def _prefetch():
                        in_copy(nxt, ins[0], sis[0]).start()

                in_copy(base, ins[b], sis[b]).wait()
                bin_chunk(ins[b], hist)

                @pl.when(p > 0)
                def _drain_prev_out():
                    out_copy(base, obs[b], sos[b]).wait()

                readout_chunk(hist, obs[b])
                out_copy(base, obs[b], sos[b]).start()
            return carry

        lax.fori_loop(0, NPAIR, pair_body, 0)

        last0 = wbase + (NCH - 2) * CB
        out_copy(last0, ob0, sem_o0).wait()
        out_copy(last0 + CB, ob1, sem_o1).wait()

    mesh = plsc.VectorSubcoreMesh(
        core_axis_name="c", subcore_axis_name="s",
        num_cores=2, num_subcores=16)
    return functools.partial(
        pl.kernel,
        mesh=mesh,
        out_type=jax.ShapeDtypeStruct((B, _NBINS, _P), jnp.float32),
        scratch_types=[
            pltpu.VMEM((CB, _S, _P), jnp.float32),
            pltpu.VMEM((CB, _S, _P), jnp.float32),
            pltpu.VMEM((CB, _NBINS, _P), jnp.float32),
            pltpu.VMEM((CB, _NBINS, _P), jnp.float32),
            pltpu.VMEM((C * _SLOTS,), jnp.float32),
            pltpu.SemaphoreType.DMA,
            pltpu.SemaphoreType.DMA,
            pltpu.SemaphoreType.DMA,
            pltpu.SemaphoreType.DMA,
        ],
        compiler_params=pltpu.CompilerParams(
            needs_layout_passes=False, use_tc_tiling_on_sc=True),
    )(sc_body)


def kernel(x):
    batch, patches, seq = x.shape
    xt = jnp.transpose(x, (0, 2, 1))
    run = _make_sc(batch, 2)
    zt = run(xt)
    return jnp.transpose(zt, (0, 2, 1))
